# Initial kernel scaffold; baseline (speedup 1.0000x reference)
import jax
import jax.numpy as jnp
from jax.experimental import pallas as pl


def kernel(inputs, expert_tokens_outer, ln1_g, ln1_b, ln2_g, ln2_b, Wq, Wkv, Wproj, bproj, moe_tokens, W1, b1, W2, b2, W3, b3, Wg, bg):
    raise NotImplementedError("write your pallas kernel here")


# speedup vs baseline: 6.1033x; 6.1033x over previous
import jax
import jax.numpy as jnp
from jax.experimental import pallas as pl
from jax.experimental.pallas import tpu as pltpu

D = 768
E = 8
H = 12
DH = D // H
HID = 2 * D
N = 2048
TOPK = 2
_HIGH = jax.lax.Precision.HIGHEST


def _ln(x, g, b):
    m = jnp.mean(x, axis=-1, keepdims=True)
    v = jnp.mean((x - m) ** 2, axis=-1, keepdims=True)
    return (x - m) / jnp.sqrt(v + 1e-5) * g + b


def _dot(a, b):
    return jax.lax.dot_general(a, b, (((1,), (0,)), ((), ())), precision=_HIGH)


def _kv_proj_kernel(x_ref, g_ref, b_ref, wkv_ref, kv_ref):
    xn = _ln(x_ref[...], g_ref[...], b_ref[...])
    kv_ref[...] = _dot(xn, wkv_ref[...])


def _attn_route_kernel(et_ref, in2_ref, g1_ref, b1_ref, g2_ref, b2_ref,
                       wq_ref, wkv_ref, wproj_ref, bproj_ref, mtok_ref,
                       kvin_ref,
                       xc_ref, xn2_ref, mask_ref, fused_ref):
    et = et_ref[...]
    xn8 = _ln(et, g1_ref[...], b1_ref[...])
    q = _dot(xn8, wq_ref[...])
    kv8 = _dot(xn8, wkv_ref[...])
    kv = jnp.concatenate([kv8, kvin_ref[...]], axis=0)
    scale = DH ** -0.5
    ao_heads = []
    for h in range(H):
        qh = q[:, h * DH:(h + 1) * DH]
        kh = kv[:, h * DH:(h + 1) * DH]
        vh = kv[:, D + h * DH:D + (h + 1) * DH]
        s = jax.lax.dot_general(qh, kh, (((1,), (1,)), ((), ())),
                                precision=_HIGH) * scale
        s = s - jnp.max(s, axis=-1, keepdims=True)
        p = jnp.exp(s)
        p = p / jnp.sum(p, axis=-1, keepdims=True)
        ao_heads.append(_dot(p, vh))
    ao = jnp.concatenate(ao_heads, axis=1)
    ao = _dot(ao, wproj_ref[...]) + bproj_ref[...]
    xc8 = et + ao
    xn2 = _ln(xc8, g2_ref[...], b2_ref[...])
    scores = jax.lax.dot_general(xn2, mtok_ref[...], (((1,), (1,)), ((), ())),
                                 precision=_HIGH)
    col = jax.lax.broadcasted_iota(jnp.int32, (E, E), 1)
    i1 = jnp.argmax(scores, axis=-1)
    oh1 = (col == i1[:, None])
    scores2 = jnp.where(oh1, -jnp.inf, scores)
    i2 = jnp.argmax(scores2, axis=-1)
    oh2 = (col == i2[:, None])
    mask = 0.5 * oh1.astype(jnp.float32) + 0.5 * oh2.astype(jnp.float32)
    xc_ref[...] = xc8
    xn2_ref[...] = xn2
    mask_ref[...] = jnp.broadcast_to(mask.T[:, :, None], (E, E, 128))
    fused_ref[...] = 0.5 * (in2_ref[0:1, :] + in2_ref[1:2, :])


def _gelu(x):
    return x * 0.5 * (1.0 + jax.lax.erf(x * (2.0 ** -0.5)))


def _moe_kernel(xn2_ref, mask_ref, xc_ref, w1_ref, b1_ref, w2_ref, b2_ref,
                w3_ref, b3_ref, wg_ref, bg_ref, ef_ref, conf_ref):
    e = pl.program_id(0)
    xn2 = xn2_ref[...]
    h1 = _gelu(_dot(xn2, w1_ref[0]) + b1_ref[0])
    h2 = _gelu(_dot(h1, w2_ref[0]) + b2_ref[0])
    o = _dot(h2, w3_ref[0]) + b3_ref[0]
    contrib = o * mask_ref[0][:, :1]

    @pl.when(e == 0)
    def _():
        ef_ref[...] = xc_ref[...] + contrib

    @pl.when(e > 0)
    def _():
        ef_ref[...] = ef_ref[...] + contrib

    @pl.when(e == E - 1)
    def _():
        ef = ef_ref[...]
        conf_ref[...] = jax.nn.sigmoid(_dot(ef, wg_ref[...]) + bg_ref[...])


def kernel(inputs, expert_tokens_outer, ln1_g, ln1_b, ln2_g, ln2_b, Wq, Wkv,
           Wproj, bproj, moe_tokens, W1, b1, W2, b2, W3, b3, Wg, bg):
    x = inputs[0]
    g1 = ln1_g.reshape(1, D)
    b1v = ln1_b.reshape(1, D)
    g2 = ln2_g.reshape(1, D)
    b2v = ln2_b.reshape(1, D)
    bproj2 = bproj.reshape(1, D)

    blk = 256
    kv = pl.pallas_call(
        _kv_proj_kernel,
        grid=(N // blk,),
        in_specs=[
            pl.BlockSpec((blk, D), lambda i: (i, 0)),
            pl.BlockSpec((1, D), lambda i: (0, 0)),
            pl.BlockSpec((1, D), lambda i: (0, 0)),
            pl.BlockSpec((D, 2 * D), lambda i: (0, 0)),
        ],
        out_specs=pl.BlockSpec((blk, 2 * D), lambda i: (i, 0)),
        out_shape=jax.ShapeDtypeStruct((N, 2 * D), jnp.float32),
        compiler_params=pltpu.CompilerParams(
            dimension_semantics=("arbitrary",)),
    )(x, g1, b1v, Wkv)

    xc8, xn2, maskp, fused = pl.pallas_call(
        _attn_route_kernel,
        out_shape=(
            jax.ShapeDtypeStruct((E, D), jnp.float32),
            jax.ShapeDtypeStruct((E, D), jnp.float32),
            jax.ShapeDtypeStruct((E, E, 128), jnp.float32),
            jax.ShapeDtypeStruct((1, D), jnp.float32),
        ),
    )(expert_tokens_outer, x[:8], g1, b1v, g2, b2v, Wq, Wkv, Wproj, bproj2,
      moe_tokens, kv)

    wg_pad = jnp.concatenate([Wg, jnp.zeros((D, 127), jnp.float32)], axis=1)
    bg_pad = jnp.broadcast_to(bg.reshape(1, 1), (1, 128))
    ef, confp = pl.pallas_call(
        _moe_kernel,
        grid=(E,),
        in_specs=[
            pl.BlockSpec((E, D), lambda e: (0, 0)),
            pl.BlockSpec((1, E, 128), lambda e: (e, 0, 0)),
            pl.BlockSpec((E, D), lambda e: (0, 0)),
            pl.BlockSpec((1, D, HID), lambda e: (e, 0, 0)),
            pl.BlockSpec((1, 1, HID), lambda e: (e, 0, 0)),
            pl.BlockSpec((1, HID, HID), lambda e: (e, 0, 0)),
            pl.BlockSpec((1, 1, HID), lambda e: (e, 0, 0)),
            pl.BlockSpec((1, HID, D), lambda e: (e, 0, 0)),
            pl.BlockSpec((1, 1, D), lambda e: (e, 0, 0)),
            pl.BlockSpec((D, 128), lambda e: (0, 0)),
            pl.BlockSpec((1, 128), lambda e: (0, 0)),
        ],
        out_specs=(
            pl.BlockSpec((E, D), lambda e: (0, 0)),
            pl.BlockSpec((E, 128), lambda e: (0, 0)),
        ),
        out_shape=(
            jax.ShapeDtypeStruct((E, D), jnp.float32),
            jax.ShapeDtypeStruct((E, 128), jnp.float32),
        ),
        compiler_params=pltpu.CompilerParams(
            dimension_semantics=("arbitrary",)),
    )(xn2, maskp, xc8, W1, b1.reshape(E, 1, HID), W2, b2.reshape(E, 1, HID),
      W3, b3.reshape(E, 1, D), wg_pad, bg_pad)

    expert_features = ef[None]
    confidence = confp[:, :1][None]
    return (expert_features, confidence, fused)
